# baseline (speedup 1.0000x reference)
import functools

import jax
import jax.numpy as jnp
from jax import lax
from jax.experimental import pallas as pl
from jax.experimental.pallas import tpu as pltpu
from jax.experimental.pallas import tpu_sc as plsc

_NC = 2
_NS = 16
_NW = _NC * _NS

_SEQ = 200
_HALF = 100
_HPAD = 104


def _pool_sc(x2, table, batch):
    b_per_w = batch // _NW
    rows_per_w = 2 * b_per_w

    mesh = plsc.VectorSubcoreMesh(core_axis_name="c", subcore_axis_name="s")

    @functools.partial(
        pl.kernel,
        out_type=jax.ShapeDtypeStruct((batch, 64), jnp.float32),
        mesh=mesh,
        scratch_types=[
            pltpu.VMEM((rows_per_w, _HPAD), jnp.int32),
            pltpu.VMEM((2 * _HPAD, 64), jnp.float32),
            pltpu.VMEM((b_per_w, 64), jnp.float32),
            pltpu.SemaphoreType.DMA,
        ],
        compiler_params=pltpu.CompilerParams(use_tc_tiling_on_sc=False),
    )
    def pool(x_hbm, tab_hbm, out_hbm, idx_v, rows_v, acc_v, sem):
        wid = lax.axis_index("s") * _NC + lax.axis_index("c")
        pltpu.sync_copy(x_hbm.at[pl.ds(wid * rows_per_w, rows_per_w)], idx_v)

        def row_body(r, _):
            cp0 = pltpu.async_copy(
                tab_hbm.at[idx_v.at[2 * r]], rows_v.at[pl.ds(0, _HPAD)], sem)
            cp1 = pltpu.async_copy(
                tab_hbm.at[idx_v.at[2 * r + 1]], rows_v.at[pl.ds(_HPAD, _HPAD)], sem)
            cp0.wait()
            cp1.wait()

            def acc_body(base):
                def body(s, carry):
                    a0, a1, a2, a3 = carry
                    a0 = a0 + rows_v[s, pl.ds(0, 16)]
                    a1 = a1 + rows_v[s, pl.ds(16, 16)]
                    a2 = a2 + rows_v[s, pl.ds(32, 16)]
                    a3 = a3 + rows_v[s, pl.ds(48, 16)]
                    return a0, a1, a2, a3
                return body

            z = jnp.zeros((16,), jnp.float32)
            acc = lax.fori_loop(0, _HALF, acc_body(0), (z, z, z, z), unroll=4)
            acc = lax.fori_loop(_HPAD, _HPAD + _HALF, acc_body(0), acc, unroll=4)
            a0, a1, a2, a3 = acc
            acc_v[r, pl.ds(0, 16)] = a0
            acc_v[r, pl.ds(16, 16)] = a1
            acc_v[r, pl.ds(32, 16)] = a2
            acc_v[r, pl.ds(48, 16)] = a3
            return _

        lax.fori_loop(0, b_per_w, row_body, None)
        pltpu.sync_copy(acc_v, out_hbm.at[pl.ds(wid * b_per_w, b_per_w)])

    return pool(x2, table)


def _mlp_body(p_ref, w1_ref, b1_ref, w2_ref, b2_ref, o_ref):
    p = p_ref[...] * (1.0 / _SEQ)
    h = jnp.dot(p, w1_ref[...], preferred_element_type=jnp.float32) + b1_ref[...]
    h = jnp.maximum(h, 0.0)
    logits = jnp.dot(h, w2_ref[...], preferred_element_type=jnp.float32) + b2_ref[...]
    m = jnp.max(logits, axis=1, keepdims=True)
    ex = jnp.exp(logits - m)
    o_ref[...] = logits - m - jnp.log(jnp.sum(ex, axis=1, keepdims=True))


def _mlp_tc(sums, W1, b1, W2, b2):
    batch, embed = sums.shape
    hidden = W1.shape[1]
    out = W2.shape[1]
    blk = 512
    return pl.pallas_call(
        _mlp_body,
        grid=(batch // blk,),
        in_specs=[
            pl.BlockSpec((blk, embed), lambda i: (i, 0)),
            pl.BlockSpec((embed, hidden), lambda i: (0, 0)),
            pl.BlockSpec((1, hidden), lambda i: (0, 0)),
            pl.BlockSpec((hidden, out), lambda i: (0, 0)),
            pl.BlockSpec((1, out), lambda i: (0, 0)),
        ],
        out_specs=pl.BlockSpec((blk, out), lambda i: (i, 0)),
        out_shape=jax.ShapeDtypeStruct((batch, out), jnp.float32),
    )(sums, W1, b1.reshape(1, hidden), W2, b2.reshape(1, out))


def kernel(x, table, W1, b1, W2, b2):
    batch, seq = x.shape
    assert seq == _SEQ and batch % _NW == 0
    x2 = jnp.pad(x.reshape(batch, 2, _HALF),
                 ((0, 0), (0, 0), (0, _HPAD - _HALF))).reshape(2 * batch, _HPAD)
    sums = _pool_sc(x2, table, batch)
    return _mlp_tc(sums, W1, b1, W2, b2)


# speedup vs baseline: 1.8281x; 1.8281x over previous
import functools

import jax
import jax.numpy as jnp
from jax import lax
from jax.experimental import pallas as pl
from jax.experimental.pallas import tpu as pltpu
from jax.experimental.pallas import tpu_sc as plsc

_NC = 2
_NS = 16
_NW = _NC * _NS

_SEQ = 200
_S0 = 96
_S1 = 104


def _pool_sc(x, table, batch):
    b_per_w = batch // _NW

    mesh = plsc.VectorSubcoreMesh(core_axis_name="c", subcore_axis_name="s")

    @functools.partial(
        pl.kernel,
        out_type=jax.ShapeDtypeStruct((batch, 64), jnp.float32),
        mesh=mesh,
        scratch_types=[
            pltpu.VMEM((b_per_w, _SEQ), jnp.int32),
            pltpu.VMEM((2, _SEQ, 64), jnp.float32),
            pltpu.VMEM((b_per_w, 64), jnp.float32),
            pltpu.SemaphoreType.DMA,
            pltpu.SemaphoreType.DMA,
        ],
        compiler_params=pltpu.CompilerParams(use_tc_tiling_on_sc=False),
    )
    def pool(x_hbm, tab_hbm, out_hbm, idx_v, rows_v, acc_v, sem0, sem1):
        wid = lax.axis_index("s") * _NC + lax.axis_index("c")
        pltpu.sync_copy(x_hbm.at[pl.ds(wid * b_per_w, b_per_w)], idx_v)

        def issue(r, buf, sem):
            pltpu.async_copy(
                tab_hbm.at[idx_v.at[r, pl.ds(0, _S0)]],
                rows_v.at[buf, pl.ds(0, _S0)], sem)
            pltpu.async_copy(
                tab_hbm.at[idx_v.at[r, pl.ds(_S0, _S1)]],
                rows_v.at[buf, pl.ds(_S0, _S1)], sem)

        def wait(buf, sem):
            pltpu.make_async_copy(
                tab_hbm.at[pl.ds(0, _SEQ)], rows_v.at[buf], sem).wait()

        def accumulate(r, buf):
            def body(s, carry):
                a0, a1, a2, a3 = carry
                a0 = a0 + rows_v[buf, s, pl.ds(0, 16)]
                a1 = a1 + rows_v[buf, s, pl.ds(16, 16)]
                a2 = a2 + rows_v[buf, s, pl.ds(32, 16)]
                a3 = a3 + rows_v[buf, s, pl.ds(48, 16)]
                return a0, a1, a2, a3

            z = jnp.zeros((16,), jnp.float32)
            a0, a1, a2, a3 = lax.fori_loop(0, _SEQ, body, (z, z, z, z),
                                           unroll=8)
            acc_v[r, pl.ds(0, 16)] = a0
            acc_v[r, pl.ds(16, 16)] = a1
            acc_v[r, pl.ds(32, 16)] = a2
            acc_v[r, pl.ds(48, 16)] = a3

        issue(0, 0, sem0)

        def pair_body(g, _):
            r0 = 2 * g
            issue(r0 + 1, 1, sem1)
            wait(0, sem0)
            accumulate(r0, 0)

            @pl.when(r0 + 2 < b_per_w)
            def _issue_next():
                issue(r0 + 2, 0, sem0)

            wait(1, sem1)
            accumulate(r0 + 1, 1)
            return _

        lax.fori_loop(0, b_per_w // 2, pair_body, None)
        pltpu.sync_copy(acc_v, out_hbm.at[pl.ds(wid * b_per_w, b_per_w)])

    return pool(x, table)


def _mlp_body(p_ref, w1_ref, b1_ref, w2_ref, b2_ref, o_ref):
    p = p_ref[...] * (1.0 / _SEQ)
    h = jnp.dot(p, w1_ref[...], preferred_element_type=jnp.float32) + b1_ref[...]
    h = jnp.maximum(h, 0.0)
    logits = jnp.dot(h, w2_ref[...], preferred_element_type=jnp.float32) + b2_ref[...]
    m = jnp.max(logits, axis=1, keepdims=True)
    ex = jnp.exp(logits - m)
    o_ref[...] = logits - m - jnp.log(jnp.sum(ex, axis=1, keepdims=True))


def _mlp_tc(sums, W1, b1, W2, b2):
    batch, embed = sums.shape
    hidden = W1.shape[1]
    out = W2.shape[1]
    blk = 512
    return pl.pallas_call(
        _mlp_body,
        grid=(batch // blk,),
        in_specs=[
            pl.BlockSpec((blk, embed), lambda i: (i, 0)),
            pl.BlockSpec((embed, hidden), lambda i: (0, 0)),
            pl.BlockSpec((1, hidden), lambda i: (0, 0)),
            pl.BlockSpec((hidden, out), lambda i: (0, 0)),
            pl.BlockSpec((1, out), lambda i: (0, 0)),
        ],
        out_specs=pl.BlockSpec((blk, out), lambda i: (i, 0)),
        out_shape=jax.ShapeDtypeStruct((batch, out), jnp.float32),
    )(sums, W1, b1.reshape(1, hidden), W2, b2.reshape(1, out))


def kernel(x, table, W1, b1, W2, b2):
    batch, seq = x.shape
    assert seq == _SEQ and batch % _NW == 0
    sums = _pool_sc(x, table, batch)
    return _mlp_tc(sums, W1, b1, W2, b2)
